# baseline (speedup 1.0000x reference)
import jax
import jax.numpy as jnp
from jax import lax
from jax.experimental import pallas as pl
from jax.experimental.pallas import tpu as pltpu
from jax.experimental.pallas import tpu_sc as plsc

MINUTE_SIZE = 60
D_MODEL = 512

_N = 4096 * 200
_NW = 32
_PER_W = _N // _NW
_CHUNK = 80
_STEPS = _PER_W // _CHUNK
_L = 16
_G = _CHUNK // _L


def _sc_kernel(x_hbm, w_hbm, out_hbm, w_tile, xbufs, idxs, rows, wsems):
    wid = lax.axis_index("s") * 2 + lax.axis_index("c")
    base0 = wid * _PER_W
    lanes = lax.iota(jnp.int32, _L)

    pltpu.sync_copy(w_hbm, w_tile)

    def do_chunk(g, b):
        base = base0 + g * _CHUNK
        pltpu.sync_copy(x_hbm.at[pl.ds(base * 4, _CHUNK * 4)], xbufs[b])
        for j in range(_G):
            flat = lanes * 4 + (j * _L * 4 + 1)
            idxs[b][pl.ds(j * _L, _L)] = plsc.load_gather(xbufs[b], [flat])
        for grp in range(_G):
            idx_vec = idxs[b][pl.ds(grp * _L, _L)]
            gbase = idx_vec * D_MODEL
            sbase = lanes * D_MODEL + grp * _L * D_MODEL

            @plsc.parallel_loop(0, D_MODEL, 1, unroll=8)
            def _(c):
                vals = plsc.load_gather(w_tile, [gbase + c])
                plsc.store_scatter(rows[b], [sbase + c], vals)

        pltpu.async_copy(
            rows[b], out_hbm.at[pl.ds(base * D_MODEL, _CHUNK * D_MODEL)],
            wsems[b])

    def wait_write(b):
        pltpu.make_async_copy(
            rows[b], out_hbm.at[pl.ds(0, _CHUNK * D_MODEL)], wsems[b]).wait()

    do_chunk(0, 0)
    do_chunk(1, 1)

    def body(h, carry):
        for b in range(2):
            g = 2 * h + b + 2
            wait_write(b)
            do_chunk(g, b)
        return carry

    lax.fori_loop(0, (_STEPS - 2) // 2, body, 0)
    wait_write(0)
    wait_write(1)


@jax.jit
def kernel(x, W):
    x2 = x.reshape(_N * 4).astype(jnp.int32)
    w2 = W.reshape(MINUTE_SIZE * D_MODEL)
    mesh = plsc.VectorSubcoreMesh(core_axis_name="c", subcore_axis_name="s")

    def body(x_hbm, w_hbm, out_hbm, w_tile, xb0, xb1, id0, id1, r0, r1,
             s0, s1):
        _sc_kernel(x_hbm, w_hbm, out_hbm, w_tile,
                   (xb0, xb1), (id0, id1), (r0, r1), (s0, s1))

    out = pl.kernel(
        body,
        mesh=mesh,
        compiler_params=pltpu.CompilerParams(needs_layout_passes=False),
        out_type=jax.ShapeDtypeStruct((_N * D_MODEL,), jnp.float32),
        scratch_types=[
            pltpu.VMEM((MINUTE_SIZE * D_MODEL,), jnp.float32),
            pltpu.VMEM((_CHUNK * 4,), jnp.int32),
            pltpu.VMEM((_CHUNK * 4,), jnp.int32),
            pltpu.VMEM((_CHUNK,), jnp.int32),
            pltpu.VMEM((_CHUNK,), jnp.int32),
            pltpu.VMEM((_CHUNK * D_MODEL,), jnp.float32),
            pltpu.VMEM((_CHUNK * D_MODEL,), jnp.float32),
            pltpu.SemaphoreType.DMA,
            pltpu.SemaphoreType.DMA,
        ],
    )(x2, w2)
    return out.reshape(4096, 200, D_MODEL)


# speedup vs baseline: 2.4373x; 2.4373x over previous
import jax
import jax.numpy as jnp
from jax import lax
from jax.experimental import pallas as pl
from jax.experimental.pallas import tpu as pltpu
from jax.experimental.pallas import tpu_sc as plsc

MINUTE_SIZE = 60
D_MODEL = 512

_N = 4096 * 200
_NW = 32
_PER_W = _N // _NW
_CHUNK = 80
_STEPS = _PER_W // _CHUNK
_L = 16
_G = _CHUNK // _L


def _sc_kernel(x_hbm, w_hbm, out_hbm, w_tile, xbufs, idxs, rows, wsems):
    wid = lax.axis_index("s") * 2 + lax.axis_index("c")
    base0 = wid * _PER_W
    lanes = lax.iota(jnp.int32, _L)

    pltpu.sync_copy(w_hbm, w_tile)

    def do_chunk(g, b):
        base = base0 + g * _CHUNK
        pltpu.sync_copy(x_hbm.at[pl.ds(base * 4, _CHUNK * 4)], xbufs[b])
        for j in range(_G):
            flat = lanes * 4 + (j * _L * 4 + 1)
            idxs[b][pl.ds(j * _L, _L)] = (
                plsc.load_gather(xbufs[b], [flat]) * D_MODEL)
        @plsc.parallel_loop(0, _CHUNK, 1, unroll=2)
        def _(r):
            off = idxs[b][pl.ds(r, _L)][0]
            for j in range(D_MODEL // _L):
                rows[b][pl.ds(r * D_MODEL + j * _L, _L)] = (
                    w_tile[pl.ds(off + j * _L, _L)])

        pltpu.async_copy(
            rows[b], out_hbm.at[pl.ds(base * D_MODEL, _CHUNK * D_MODEL)],
            wsems[b])

    def wait_write(b):
        pltpu.make_async_copy(
            rows[b], out_hbm.at[pl.ds(0, _CHUNK * D_MODEL)], wsems[b]).wait()

    def body(h, carry):
        for b in range(2):
            g = 2 * h + b

            @pl.when(g >= 2)
            def _():
                wait_write(b)

            do_chunk(g, b)
        return carry

    lax.fori_loop(0, _STEPS // 2, body, 0)
    wait_write(0)
    wait_write(1)


@jax.jit
def kernel(x, W):
    x2 = x.reshape(_N * 4).astype(jnp.int32)
    w2 = W.reshape(MINUTE_SIZE * D_MODEL)
    mesh = plsc.VectorSubcoreMesh(core_axis_name="c", subcore_axis_name="s")

    def body(x_hbm, w_hbm, out_hbm, w_tile, xb0, xb1, id0, id1, r0, r1,
             s0, s1):
        _sc_kernel(x_hbm, w_hbm, out_hbm, w_tile,
                   (xb0, xb1), (id0, id1), (r0, r1), (s0, s1))

    out = pl.kernel(
        body,
        mesh=mesh,
        compiler_params=pltpu.CompilerParams(needs_layout_passes=False),
        out_type=jax.ShapeDtypeStruct((_N * D_MODEL,), jnp.float32),
        scratch_types=[
            pltpu.VMEM((MINUTE_SIZE * D_MODEL,), jnp.float32),
            pltpu.VMEM((_CHUNK * 4,), jnp.int32),
            pltpu.VMEM((_CHUNK * 4,), jnp.int32),
            pltpu.VMEM((_CHUNK + _L,), jnp.int32),
            pltpu.VMEM((_CHUNK + _L,), jnp.int32),
            pltpu.VMEM((_CHUNK * D_MODEL,), jnp.float32),
            pltpu.VMEM((_CHUNK * D_MODEL,), jnp.float32),
            pltpu.SemaphoreType.DMA,
            pltpu.SemaphoreType.DMA,
        ],
    )(x2, w2)
    return out.reshape(4096, 200, D_MODEL)
